# Initial kernel scaffold; baseline (speedup 1.0000x reference)
import jax
import jax.numpy as jnp
from jax.experimental import pallas as pl


def kernel(x, t, pos, edge_index_local, edge_index_global, batch, params):
    raise NotImplementedError("write your pallas kernel here")


# speedup vs baseline: 10.0129x; 10.0129x over previous
import functools

import jax
import jax.numpy as jnp
from jax import lax
from jax.experimental import pallas as pl
from jax.experimental.pallas import tpu as pltpu
from jax.experimental.pallas import tpu_sc as plsc

NND = 10000
NED = 160000
NGR = 256
SDIM = 128
VDIM = 64
RBFD = 32
ATY = 16
BTY = 5
CUT = 7.5

E_PAD = 163840
N_ACC = 10240
DUMMY = NND
NC, NS = 2, 16
CHUNK = 128

PREC = lax.Precision.HIGHEST
F32 = jnp.float32


def _mesh():
    return plsc.VectorSubcoreMesh(core_axis_name="c", subcore_axis_name="s")


_SC_PARAMS = pltpu.CompilerParams(use_tc_tiling_on_sc=False)


def _sc_gather(table, idx0, idx1, d):
    per_tile = E_PAD // (NC * NS)
    n_chunks = per_tile // CHUNK

    @functools.partial(
        pl.kernel, mesh=_mesh(),
        out_type=[jax.ShapeDtypeStruct((E_PAD, d), F32),
                  jax.ShapeDtypeStruct((E_PAD, d), F32)],
        scratch_types=[pltpu.VMEM((2, n_chunks, CHUNK), jnp.int32),
                       pltpu.VMEM((CHUNK, d), F32),
                       pltpu.VMEM((CHUNK, d), F32),
                       pltpu.VMEM((CHUNK, d), F32),
                       pltpu.VMEM((CHUNK, d), F32),
                       pltpu.SemaphoreType.DMA,
                       pltpu.SemaphoreType.DMA,
                       pltpu.SemaphoreType.DMA,
                       pltpu.SemaphoreType.DMA,
                       pltpu.SemaphoreType.DMA,
                       pltpu.SemaphoreType.DMA,
                       pltpu.SemaphoreType.DMA,
                       pltpu.SemaphoreType.DMA],
        compiler_params=_SC_PARAMS,
    )
    def k(table_hbm, i0_hbm, i1_hbm, o0_hbm, o1_hbm, idxv,
          ra0, ra1, rb0, rb1, ga0, ga1, gb0, gb1, oa0, oa1, ob0, ob1):
        wid = lax.axis_index("s") * NC + lax.axis_index("c")
        base = wid * per_tile
        pltpu.sync_copy(i0_hbm.at[pl.ds(wid * n_chunks, n_chunks)],
                        idxv.at[0])
        pltpu.sync_copy(i1_hbm.at[pl.ds(wid * n_chunks, n_chunks)],
                        idxv.at[1])

        bufs = {0: (ra0, rb0), 1: (ra1, rb1)}
        gsems = {0: (ga0, gb0), 1: (ga1, gb1)}
        osems = {0: (oa0, ob0), 1: (oa1, ob1)}
        outs = {0: o0_hbm, 1: o1_hbm}
        gh = {}
        oh = {}

        def start_gather(s_, j):
            b = s_, j % 2
            gh[b] = pltpu.async_copy(table_hbm.at[idxv.at[s_, j]],
                                     bufs[s_][j % 2], gsems[s_][j % 2])

        for s_ in (0, 1):
            start_gather(s_, 0)
            start_gather(s_, 1)
        for j in range(n_chunks):
            for s_ in (0, 1):
                p_ = j % 2
                gh[(s_, p_)].wait()
                oh[(s_, p_)] = pltpu.async_copy(
                    bufs[s_][p_],
                    outs[s_].at[pl.ds(base + j * CHUNK, CHUNK)],
                    osems[s_][p_])
            if j + 2 < n_chunks:
                for s_ in (0, 1):
                    p_ = j % 2
                    oh[(s_, p_)].wait()
                    start_gather(s_, j + 2)
        for s_ in (0, 1):
            oh[(s_, (n_chunks - 2) % 2)].wait()
            oh[(s_, (n_chunks - 1) % 2)].wait()

    return k(table, idx0, idx1)


def _sc_scatter_pair(p0, p1, dst_sc, zeros160):
    rows_per_tile = N_ACC // NS
    per_tile = E_PAD // NS
    n_chunks = per_tile // CHUNK

    @functools.partial(
        pl.kernel, mesh=_mesh(),
        out_type=[jax.ShapeDtypeStruct((N_ACC, 160), F32),
                  jax.ShapeDtypeStruct((N_ACC, 160), F32)],
        scratch_types=[pltpu.VMEM((1, CHUNK), jnp.int32),
                       pltpu.VMEM((CHUNK, 160), F32),
                       pltpu.VMEM_SHARED((N_ACC, 160), F32)],
        compiler_params=_SC_PARAMS,
    )
    def k(p0_hbm, p1_hbm, d_hbm, z_hbm, a0_hbm, a1_hbm, ibuf, pbuf, acc):
        cid = lax.axis_index("c")
        sid = lax.axis_index("s")
        r0 = sid * rows_per_tile
        base = sid * per_tile
        pltpu.sync_copy(z_hbm.at[pl.ds(r0, rows_per_tile)],
                        acc.at[pl.ds(r0, rows_per_tile)])
        plsc.subcore_barrier()

        def body(p_hbm):
            @pl.loop(0, n_chunks)
            def _(j):
                pltpu.sync_copy(d_hbm.at[pl.ds(sid * n_chunks + j, 1)],
                                ibuf)
                pltpu.sync_copy(p_hbm.at[pl.ds(base + j * CHUNK, CHUNK)],
                                pbuf)
                pltpu.sync_copy(pbuf, acc.at[ibuf.at[0]], add=True)

        @pl.when(cid == 0)
        def _():
            body(p0_hbm)

        @pl.when(cid == 1)
        def _():
            body(p1_hbm)

        plsc.subcore_barrier()

        @pl.when(cid == 0)
        def _():
            pltpu.sync_copy(acc.at[pl.ds(r0, rows_per_tile)],
                            a0_hbm.at[pl.ds(r0, rows_per_tile)])

        @pl.when(cid == 1)
        def _():
            pltpu.sync_copy(acc.at[pl.ds(r0, rows_per_tile)],
                            a1_hbm.at[pl.ds(r0, rows_per_tile)])

    return k(p0, p1, dst_sc, zeros160)


def _sc_counts(dl_sc, dg_sc, zeros16):
    rows_per_tile = N_ACC // NS
    per_tile = E_PAD // NS
    n_chunks = per_tile // CHUNK

    @functools.partial(
        pl.kernel, mesh=_mesh(),
        out_type=[jax.ShapeDtypeStruct((N_ACC, 16), F32),
                  jax.ShapeDtypeStruct((N_ACC, 16), F32)],
        scratch_types=[pltpu.VMEM((CHUNK,), jnp.int32),
                       pltpu.VMEM((CHUNK, 16), F32),
                       pltpu.VMEM_SHARED((N_ACC, 16), F32),
                       pltpu.SemaphoreType.DMA],
        compiler_params=_SC_PARAMS,
    )
    def k(dl_hbm, dg_hbm, z_hbm, cl_hbm, cg_hbm, ibuf, ones, acc, sem):
        cid = lax.axis_index("c")
        sid = lax.axis_index("s")

        @pl.loop(0, CHUNK)
        def _(r):
            ones[r] = jnp.ones((16,), F32)

        r0 = sid * rows_per_tile
        pltpu.sync_copy(z_hbm.at[pl.ds(r0, rows_per_tile)],
                        acc.at[pl.ds(r0, rows_per_tile)])
        plsc.subcore_barrier()

        def body(d_hbm):
            @pl.loop(0, n_chunks)
            def _(j):
                pltpu.sync_copy(d_hbm.at[sid * n_chunks + j], ibuf)
                pltpu.sync_copy(ones, acc.at[ibuf], add=True)

        @pl.when(cid == 0)
        def _():
            body(dl_hbm)

        @pl.when(cid == 1)
        def _():
            body(dg_hbm)

        plsc.subcore_barrier()

        @pl.when(cid == 0)
        def _():
            pltpu.sync_copy(acc.at[pl.ds(r0, rows_per_tile)],
                            cl_hbm.at[pl.ds(r0, rows_per_tile)])

        @pl.when(cid == 1)
        def _():
            pltpu.sync_copy(acc.at[pl.ds(r0, rows_per_tile)],
                            cg_hbm.at[pl.ds(r0, rows_per_tile)])

    return k(dl_sc, dg_sc, zeros16)


def _tc_seg_sums(pos8, batch2d):
    BLK = 2000
    grid = NND // BLK

    def body(pos_ref, b_ref, sum_ref, cnt_ref):
        i = pl.program_id(0)
        iota = lax.broadcasted_iota(jnp.int32, (NGR, BLK), 0)
        mask = (b_ref[0] == iota).astype(F32)
        psum = jnp.dot(mask, pos_ref[...], preferred_element_type=F32,
                       precision=PREC)
        csum = jnp.broadcast_to(jnp.sum(mask, axis=1, keepdims=True),
                                (NGR, 8))

        @pl.when(i == 0)
        def _():
            sum_ref[...] = jnp.zeros((NGR, 8), F32)
            cnt_ref[...] = jnp.zeros((NGR, 8), F32)

        sum_ref[...] += psum
        cnt_ref[...] += csum

    return pl.pallas_call(
        body,
        grid=(grid,),
        in_specs=[pl.BlockSpec((BLK, 8), lambda i: (i, 0)),
                  pl.BlockSpec((1, 1, BLK), lambda i: (i, 0, 0))],
        out_specs=[pl.BlockSpec((NGR, 8), lambda i: (0, 0)),
                   pl.BlockSpec((NGR, 8), lambda i: (0, 0))],
        out_shape=[jax.ShapeDtypeStruct((NGR, 8), F32),
                   jax.ShapeDtypeStruct((NGR, 8), F32)],
        compiler_params=pltpu.CompilerParams(
            dimension_semantics=("arbitrary",)),
    )(pos8, batch2d)


def _tc_prelude(x, t8, pos8, batch2d, possum, cntg, wt8, bt, wa, ba, wat,
                bat):
    BLK = 2000
    grid = NND // BLK

    def body(x_ref, t_ref, pos_ref, b_ref, ps_ref, cg_ref, wt_ref, bt_ref,
             wa_ref, ba_ref, wat_ref, bat_ref, posc8_ref, posc16_ref,
             s0_ref):
        iota = lax.broadcasted_iota(jnp.int32, (NGR, BLK), 0)
        mask = (b_ref[0] == iota).astype(F32)
        inv = 1.0 / jnp.maximum(cg_ref[...][:, 0:1], 1.0)
        mean = ps_ref[...] * inv
        posc = pos_ref[...] - lax.dot_general(
            mask, mean, (((0,), (0,)), ((), ())),
            preferred_element_type=F32, precision=PREC)
        posc8_ref[...] = posc
        posc16_ref[...] = jnp.concatenate(
            [posc, jnp.zeros((BLK, 8), F32)], axis=1)
        tn8 = lax.dot_general(mask, t_ref[...], (((0,), (0,)), ((), ())),
                              preferred_element_type=F32, precision=PREC)
        ta = jnp.dot(tn8, wt_ref[...], preferred_element_type=F32,
                     precision=PREC) + bt_ref[...]
        sa = jnp.dot(x_ref[...], wa_ref[...], preferred_element_type=F32,
                     precision=PREC) + ba_ref[...]
        s0_ref[...] = jnp.dot(sa + ta, wat_ref[...],
                              preferred_element_type=F32,
                              precision=PREC) + bat_ref[...]

    return pl.pallas_call(
        body,
        grid=(grid,),
        in_specs=[pl.BlockSpec((BLK, ATY), lambda i: (i, 0)),
                  pl.BlockSpec((NGR, 8), lambda i: (0, 0)),
                  pl.BlockSpec((BLK, 8), lambda i: (i, 0)),
                  pl.BlockSpec((1, 1, BLK), lambda i: (i, 0, 0)),
                  pl.BlockSpec((NGR, 8), lambda i: (0, 0)),
                  pl.BlockSpec((NGR, 8), lambda i: (0, 0)),
                  pl.BlockSpec((8, SDIM), lambda i: (0, 0)),
                  pl.BlockSpec((1, SDIM), lambda i: (0, 0)),
                  pl.BlockSpec((ATY, SDIM), lambda i: (0, 0)),
                  pl.BlockSpec((1, SDIM), lambda i: (0, 0)),
                  pl.BlockSpec((SDIM, SDIM), lambda i: (0, 0)),
                  pl.BlockSpec((1, SDIM), lambda i: (0, 0))],
        out_specs=[pl.BlockSpec((BLK, 8), lambda i: (i, 0)),
                   pl.BlockSpec((BLK, 16), lambda i: (i, 0)),
                   pl.BlockSpec((BLK, SDIM), lambda i: (i, 0))],
        out_shape=[jax.ShapeDtypeStruct((NND, 8), F32),
                   jax.ShapeDtypeStruct((NND, 16), F32),
                   jax.ShapeDtypeStruct((NND, SDIM), F32)],
        compiler_params=pltpu.CompilerParams(
            dimension_semantics=("parallel",)),
    )(x, t8, pos8, batch2d, possum, cntg, wt8, bt, wa, ba, wat, bat)


def _tc_attrs(ps16, pd16):
    BLK = 2048
    grid = E_PAD // BLK

    def body(ps_ref, pd_ref, rbfa_ref, rn_ref):
        ps = ps_ref[...][:, :8]
        pd = pd_ref[...][:, :8]
        r = pd - ps
        d2 = jnp.sum(r * r, axis=1, keepdims=True)
        d = jnp.sqrt(jnp.clip(d2, 1e-6, None))
        rn_ref[...] = r / d
        a = jnp.sum(pd * ps, axis=1, keepdims=True)
        mus = lax.broadcasted_iota(jnp.int32, (1, RBFD), 1).astype(F32) * (
            CUT / (RBFD - 1))
        gamma = (CUT / RBFD) ** 2
        rbf = jnp.exp(-((d - mus) ** 2) / gamma)
        rbfa_ref[...] = jnp.concatenate(
            [rbf, a, jnp.zeros((BLK, 31), F32)], axis=1)

    return pl.pallas_call(
        body,
        grid=(grid,),
        in_specs=[pl.BlockSpec((BLK, 16), lambda i: (i, 0)),
                  pl.BlockSpec((BLK, 16), lambda i: (i, 0))],
        out_specs=[pl.BlockSpec((BLK, 64), lambda i: (i, 0)),
                   pl.BlockSpec((BLK, 8), lambda i: (i, 0))],
        out_shape=[jax.ShapeDtypeStruct((E_PAD, 64), F32),
                   jax.ShapeDtypeStruct((E_PAD, 8), F32)],
        compiler_params=pltpu.CompilerParams(
            dimension_semantics=("parallel",)),
    )(ps16, pd16)


def _tc_edge_mm(sd, ss, rbfa, rn, w1, w2, w3, b2):
    BLK = 2048
    grid = E_PAD // BLK

    def body(sd_ref, ss_ref, rb_ref, rn_ref, w1_ref, w2_ref, w3_ref, b_ref,
             p0_ref, p1_ref):
        f = (jnp.dot(sd_ref[...], w1_ref[...], preferred_element_type=F32,
                     precision=PREC)
             + jnp.dot(ss_ref[...], w2_ref[...], preferred_element_type=F32,
                       precision=PREC)
             + jnp.dot(rb_ref[...], w3_ref[...], preferred_element_type=F32,
                       precision=PREC)
             + b_ref[...])
        m = f * jax.nn.sigmoid(f)
        ms = m[:, :SDIM]
        mv = m[:, SDIM:]
        rn = rn_ref[...]
        mv0 = mv * rn[:, 0:1]
        mv1 = mv * rn[:, 1:2]
        mv2 = mv * rn[:, 2:3]
        p0_ref[...] = jnp.concatenate([ms, mv0[:, :32]], axis=1)
        p1_ref[...] = jnp.concatenate([mv0[:, 32:], mv1, mv2], axis=1)

    return pl.pallas_call(
        body,
        grid=(grid,),
        in_specs=[pl.BlockSpec((BLK, SDIM), lambda i: (i, 0)),
                  pl.BlockSpec((BLK, SDIM), lambda i: (i, 0)),
                  pl.BlockSpec((BLK, 64), lambda i: (i, 0)),
                  pl.BlockSpec((BLK, 8), lambda i: (i, 0)),
                  pl.BlockSpec((SDIM, 192), lambda i: (0, 0)),
                  pl.BlockSpec((SDIM, 192), lambda i: (0, 0)),
                  pl.BlockSpec((64, 192), lambda i: (0, 0)),
                  pl.BlockSpec((1, 192), lambda i: (0, 0))],
        out_specs=[pl.BlockSpec((BLK, 160), lambda i: (i, 0)),
                   pl.BlockSpec((BLK, 160), lambda i: (i, 0))],
        out_shape=[jax.ShapeDtypeStruct((E_PAD, 160), F32),
                   jax.ShapeDtypeStruct((E_PAD, 160), F32)],
        compiler_params=pltpu.CompilerParams(
            dimension_semantics=("parallel",)),
    )(sd, ss, rbfa, rn, w1, w2, w3, b2)


def _tc_update(s, v, a0, a1, cnt):
    BLK = 1000
    grid = NND // BLK

    def body(s_ref, v_ref, a0_ref, a1_ref, c_ref, so_ref, vo_ref):
        inv = 1.0 / jnp.maximum(c_ref[...][:, 0:1], 1.0)
        a0v = a0_ref[...]
        so_ref[...] = s_ref[...] + a0v[:, :SDIM] * inv
        vo_ref[...] = v_ref[...] + jnp.concatenate(
            [a0v[:, SDIM:], a1_ref[...]], axis=1) * inv

    return pl.pallas_call(
        body,
        grid=(grid,),
        in_specs=[pl.BlockSpec((BLK, SDIM), lambda i: (i, 0)),
                  pl.BlockSpec((BLK, 192), lambda i: (i, 0)),
                  pl.BlockSpec((BLK, 160), lambda i: (i, 0)),
                  pl.BlockSpec((BLK, 160), lambda i: (i, 0)),
                  pl.BlockSpec((BLK, 16), lambda i: (i, 0))],
        out_specs=[pl.BlockSpec((BLK, SDIM), lambda i: (i, 0)),
                   pl.BlockSpec((BLK, 192), lambda i: (i, 0))],
        out_shape=[jax.ShapeDtypeStruct((NND, SDIM), F32),
                   jax.ShapeDtypeStruct((NND, 192), F32)],
        compiler_params=pltpu.CompilerParams(
            dimension_semantics=("parallel",)),
    )(s, v, a0, a1, cnt)


def _tc_head_a(s, wsh, bsh2, wa, ba2, wf):
    BLK = 2000
    grid = NND // BLK

    def body(s_ref, wsh_ref, bsh_ref, wa_ref, ba_ref, wf_ref,
             atoms_ref, g_ref):
        h = (jnp.dot(s_ref[...], wsh_ref[...], preferred_element_type=F32,
                     precision=PREC) + bsh_ref[...])
        s2 = h * jax.nn.sigmoid(h)
        atoms_ref[...] = jnp.dot(s2, wa_ref[...], preferred_element_type=F32,
                                 precision=PREC) + ba_ref[...]
        g_ref[...] = jnp.dot(s2, wf_ref[...], preferred_element_type=F32,
                             precision=PREC)

    return pl.pallas_call(
        body,
        grid=(grid,),
        in_specs=[pl.BlockSpec((BLK, SDIM), lambda i: (i, 0)),
                  pl.BlockSpec((SDIM, SDIM), lambda i: (0, 0)),
                  pl.BlockSpec((1, SDIM), lambda i: (0, 0)),
                  pl.BlockSpec((SDIM, ATY), lambda i: (0, 0)),
                  pl.BlockSpec((1, ATY), lambda i: (0, 0)),
                  pl.BlockSpec((SDIM, SDIM), lambda i: (0, 0))],
        out_specs=[pl.BlockSpec((BLK, ATY), lambda i: (i, 0)),
                   pl.BlockSpec((BLK, SDIM), lambda i: (i, 0))],
        out_shape=[jax.ShapeDtypeStruct((NND, ATY), F32),
                   jax.ShapeDtypeStruct((NND, SDIM), F32)],
        compiler_params=pltpu.CompilerParams(
            dimension_semantics=("parallel",)),
    )(s, wsh, bsh2, wa, ba2, wf)


def _tc_vhead_sums(v, batch2d, wc_big):
    BLK = 2000
    grid = NND // BLK

    def body(v_ref, b_ref, wc_ref, cp0_ref, sum_ref):
        i = pl.program_id(0)
        cp0 = jnp.dot(v_ref[...], wc_ref[...], preferred_element_type=F32,
                      precision=PREC)
        cp0_ref[...] = cp0
        iota = lax.broadcasted_iota(jnp.int32, (NGR, BLK), 0)
        mask = (b_ref[0] == iota).astype(F32)
        psum = jnp.dot(mask, cp0, preferred_element_type=F32,
                       precision=PREC)

        @pl.when(i == 0)
        def _():
            sum_ref[...] = jnp.zeros((NGR, 8), F32)

        sum_ref[...] += psum

    return pl.pallas_call(
        body,
        grid=(grid,),
        in_specs=[pl.BlockSpec((BLK, 192), lambda i: (i, 0)),
                  pl.BlockSpec((1, 1, BLK), lambda i: (i, 0, 0)),
                  pl.BlockSpec((192, 8), lambda i: (0, 0))],
        out_specs=[pl.BlockSpec((BLK, 8), lambda i: (i, 0)),
                   pl.BlockSpec((NGR, 8), lambda i: (0, 0))],
        out_shape=[jax.ShapeDtypeStruct((NND, 8), F32),
                   jax.ShapeDtypeStruct((NGR, 8), F32)],
        compiler_params=pltpu.CompilerParams(
            dimension_semantics=("arbitrary",)),
    )(v, batch2d, wc_big)


def _tc_coords(cp0, posc8, batch2d, cpsum, cntg):
    BLK = 2000
    grid = NND // BLK

    def body(cp_ref, pc_ref, b_ref, ps_ref, cg_ref, c8_ref, c16_ref):
        iota = lax.broadcasted_iota(jnp.int32, (NGR, BLK), 0)
        mask = (b_ref[0] == iota).astype(F32)
        inv = 1.0 / jnp.maximum(cg_ref[...][:, 0:1], 1.0)
        mean = ps_ref[...] * inv
        coords = pc_ref[...] + cp_ref[...] - lax.dot_general(
            mask, mean, (((0,), (0,)), ((), ())),
            preferred_element_type=F32, precision=PREC)
        c8_ref[...] = coords
        c16_ref[...] = jnp.concatenate(
            [coords, jnp.zeros((BLK, 8), F32)], axis=1)

    return pl.pallas_call(
        body,
        grid=(grid,),
        in_specs=[pl.BlockSpec((BLK, 8), lambda i: (i, 0)),
                  pl.BlockSpec((BLK, 8), lambda i: (i, 0)),
                  pl.BlockSpec((1, 1, BLK), lambda i: (i, 0, 0)),
                  pl.BlockSpec((NGR, 8), lambda i: (0, 0)),
                  pl.BlockSpec((NGR, 8), lambda i: (0, 0))],
        out_specs=[pl.BlockSpec((BLK, 8), lambda i: (i, 0)),
                   pl.BlockSpec((BLK, 16), lambda i: (i, 0))],
        out_shape=[jax.ShapeDtypeStruct((NND, 8), F32),
                   jax.ShapeDtypeStruct((NND, 16), F32)],
        compiler_params=pltpu.CompilerParams(
            dimension_semantics=("parallel",)),
    )(cp0, posc8, batch2d, cpsum, cntg)


def _tc_bond(gi, gj, ci, cj, wdd, bb0, wb1p, bb1p):
    BLK = 2048
    grid = E_PAD // BLK

    def body(gi_ref, gj_ref, ci_ref, cj_ref, wdd_ref, bb0_ref, wb1_ref,
             bb1_ref, out_ref):
        diff = ci_ref[...] - cj_ref[...]
        dd2 = jnp.sum(diff * diff, axis=1, keepdims=True)
        dd = jnp.sqrt(jnp.clip(dd2, 1e-12, None))
        h = gi_ref[...] + gj_ref[...] + dd * wdd_ref[...] + bb0_ref[...]
        h = h * jax.nn.sigmoid(h)
        out_ref[...] = jnp.dot(h, wb1_ref[...], preferred_element_type=F32,
                               precision=PREC) + bb1_ref[...]

    return pl.pallas_call(
        body,
        grid=(grid,),
        in_specs=[pl.BlockSpec((BLK, SDIM), lambda i: (i, 0)),
                  pl.BlockSpec((BLK, SDIM), lambda i: (i, 0)),
                  pl.BlockSpec((BLK, 16), lambda i: (i, 0)),
                  pl.BlockSpec((BLK, 16), lambda i: (i, 0)),
                  pl.BlockSpec((1, SDIM), lambda i: (0, 0)),
                  pl.BlockSpec((1, SDIM), lambda i: (0, 0)),
                  pl.BlockSpec((SDIM, 8), lambda i: (0, 0)),
                  pl.BlockSpec((1, 8), lambda i: (0, 0))],
        out_specs=[pl.BlockSpec((BLK, 8), lambda i: (i, 0))],
        out_shape=[jax.ShapeDtypeStruct((E_PAD, 8), F32)],
        compiler_params=pltpu.CompilerParams(
            dimension_semantics=("parallel",)),
    )(gi, gj, ci, cj, wdd, bb0, wb1p, bb1p)[0]


def kernel(x, t, pos, edge_index_local, edge_index_global, batch, params):
    p = params
    src_l = edge_index_local[0]
    dst_l = edge_index_local[1]
    src_g = edge_index_global[0]
    dst_g = edge_index_global[1]
    pad_e = E_PAD - NED

    def pad0(a):
        return jnp.concatenate([a.astype(jnp.int32),
                                jnp.zeros((pad_e,), jnp.int32)])

    def padd(a):
        return jnp.concatenate([a.astype(jnp.int32),
                                jnp.full((pad_e,), DUMMY, jnp.int32)])

    src_l_g = pad0(src_l).reshape(E_PAD // CHUNK, CHUNK)
    dst_l_g = pad0(dst_l).reshape(E_PAD // CHUNK, CHUNK)
    src_g_g = pad0(src_g).reshape(E_PAD // CHUNK, CHUNK)
    dst_g_g = pad0(dst_g).reshape(E_PAD // CHUNK, CHUNK)
    dst_l_s = padd(dst_l).reshape(E_PAD // CHUNK, CHUNK)
    dst_g_s = padd(dst_g).reshape(E_PAD // CHUNK, CHUNK)

    pos8 = jnp.pad(pos, ((0, 0), (0, 5)))
    t8 = jnp.pad(t, ((0, 0), (0, 7)))
    batch2d = batch.astype(jnp.int32).reshape(NND // 2000, 1, 2000)
    wt8 = jnp.pad(p['W_time'], ((0, 7), (0, 0)))
    bt2 = p['b_time'].reshape(1, SDIM)
    ba2 = p['b_atom'].reshape(1, SDIM)
    bat2 = p['b_at'].reshape(1, SDIM)
    zeros160 = jnp.zeros((N_ACC, 160), F32)
    zeros16 = jnp.zeros((N_ACC, 16), F32)

    possum, cntg = _tc_seg_sums(pos8, batch2d)
    posc8, posc16, s = _tc_prelude(
        x, t8, pos8, batch2d, possum, cntg, wt8, bt2, p['W_atom'], ba2,
        p['W_at'], bat2)

    cnt_l, cnt_g = _sc_counts(dst_l_s, dst_g_s, zeros16)

    ps_l, pd_l = _sc_gather(posc16, src_l_g, dst_l_g, 16)
    ps_g, pd_g = _sc_gather(posc16, src_g_g, dst_g_g, 16)
    rbfa_l, rn_l = _tc_attrs(ps_l, pd_l)
    rbfa_g, rn_g = _tc_attrs(ps_g, pd_g)

    v = jnp.zeros((NND, 192), F32)
    for l in range(3):
        lp = p['layers'][l]
        for (dg, sg, ds, rbfa, rn, cnt, w, b) in (
                (dst_l_g, src_l_g, dst_l_s, rbfa_l, rn_l, cnt_l,
                 lp['Wl'], lp['bl']),
                (dst_g_g, src_g_g, dst_g_s, rbfa_g, rn_g, cnt_g,
                 lp['Wg'], lp['bg'])):
            sdst, ssrc = _sc_gather(s, dg, sg, SDIM)
            w1 = w[:SDIM]
            w2 = w[SDIM:2 * SDIM]
            w3 = jnp.pad(w[2 * SDIM:], ((0, 31), (0, 0)))
            b2 = b.reshape(1, -1)
            p0, p1 = _tc_edge_mm(sdst, ssrc, rbfa, rn, w1, w2, w3, b2)
            a0, a1 = _sc_scatter_pair(p0, p1, ds, zeros160)
            s, v = _tc_update(s, v, a0, a1, cnt)

    wf = p['W_b0'][:SDIM]
    wdd = p['W_b0'][SDIM:SDIM + 1]
    bsh2 = p['b_sh'].reshape(1, SDIM)
    bab2 = p['b_a'].reshape(1, ATY)
    atoms16, gtab = _tc_head_a(s, p['W_sh'], bsh2, p['W_a'], bab2, wf)

    wc = p['W_c'][:, 0]
    wc_big = jnp.zeros((192, 8), F32)
    wc_big = wc_big.at[0:64, 0].set(wc)
    wc_big = wc_big.at[64:128, 1].set(wc)
    wc_big = wc_big.at[128:192, 2].set(wc)
    cp0, cpsum = _tc_vhead_sums(v, batch2d, wc_big)
    coords8, coords16 = _tc_coords(cp0, posc8, batch2d, cpsum, cntg)

    g_j, g_i = _sc_gather(gtab, src_g_g, dst_g_g, SDIM)
    c_j, c_i = _sc_gather(coords16, src_g_g, dst_g_g, 16)

    bb0 = p['b_b0'].reshape(1, SDIM)
    wb1p = jnp.pad(p['W_b1'], ((0, 0), (0, 8 - BTY)))
    bb1p = jnp.pad(p['b_b1'], ((0, 8 - BTY))).reshape(1, 8)
    bonds8 = _tc_bond(g_i, g_j, c_i, c_j, wdd, bb0, wb1p, bb1p)

    return coords8[:, :3], atoms16, bonds8[:NED, :BTY]


# speedup vs baseline: 13.5201x; 1.3503x over previous
import functools

import jax
import jax.numpy as jnp
from jax import lax
from jax.experimental import pallas as pl
from jax.experimental.pallas import tpu as pltpu
from jax.experimental.pallas import tpu_sc as plsc

NND = 10000
NED = 160000
NGR = 256
SDIM = 128
VDIM = 64
RBFD = 32
ATY = 16
BTY = 5
CUT = 7.5

E_PAD = 163840
N_ACC = 10240
DUMMY = NND
NC, NS = 2, 16
CHUNK = 128

PREC = lax.Precision.HIGHEST
F32 = jnp.float32


def _mesh():
    return plsc.VectorSubcoreMesh(core_axis_name="c", subcore_axis_name="s")


_SC_PARAMS = pltpu.CompilerParams(use_tc_tiling_on_sc=False)


def _sc_gather(table, idx0, idx1, d):
    per_tile = E_PAD // (NC * NS)
    n_chunks = per_tile // CHUNK
    rows = table.shape[0]
    rows_pc = rows // NS

    @functools.partial(
        pl.kernel, mesh=_mesh(),
        out_type=[jax.ShapeDtypeStruct((E_PAD, d), F32),
                  jax.ShapeDtypeStruct((E_PAD, d), F32)],
        scratch_types=[pltpu.VMEM((2, n_chunks, CHUNK), jnp.int32),
                       pltpu.VMEM((CHUNK, d), F32),
                       pltpu.VMEM((CHUNK, d), F32),
                       pltpu.VMEM_SHARED((rows, d), F32),
                       pltpu.SemaphoreType.DMA,
                       pltpu.SemaphoreType.DMA,
                       pltpu.SemaphoreType.DMA,
                       pltpu.SemaphoreType.DMA],
        compiler_params=_SC_PARAMS,
    )
    def k(table_hbm, i0_hbm, i1_hbm, o0_hbm, o1_hbm, idxv, r0, r1, tab,
          g0, g1, w0, w1):
        wid = lax.axis_index("s") * NC + lax.axis_index("c")
        sid = lax.axis_index("s")
        base = wid * per_tile
        pltpu.sync_copy(i0_hbm.at[pl.ds(wid * n_chunks, n_chunks)],
                        idxv.at[0])
        pltpu.sync_copy(i1_hbm.at[pl.ds(wid * n_chunks, n_chunks)],
                        idxv.at[1])
        pltpu.sync_copy(table_hbm.at[pl.ds(sid * rows_pc, rows_pc)],
                        tab.at[pl.ds(sid * rows_pc, rows_pc)])
        plsc.subcore_barrier()

        bufs = (r0, r1)
        gsems = (g0, g1)
        wsems = (w0, w1)
        outs = (o0_hbm, o1_hbm)
        gh = {}
        wh = {}
        for j in range(n_chunks):
            for s_ in (0, 1):
                if j > 0:
                    wh[s_].wait()
                gh[s_] = pltpu.async_copy(tab.at[idxv.at[s_, j]],
                                          bufs[s_], gsems[s_])
            for s_ in (0, 1):
                gh[s_].wait()
                wh[s_] = pltpu.async_copy(
                    bufs[s_],
                    outs[s_].at[pl.ds(base + j * CHUNK, CHUNK)],
                    wsems[s_])
        wh[0].wait()
        wh[1].wait()

    return k(table, idx0, idx1)


def _sc_scatter_pair(p0, p1, dst_sc, zeros160):
    rows_per_tile = N_ACC // NS
    per_tile = E_PAD // NS
    n_chunks = per_tile // CHUNK

    @functools.partial(
        pl.kernel, mesh=_mesh(),
        out_type=[jax.ShapeDtypeStruct((N_ACC, 160), F32),
                  jax.ShapeDtypeStruct((N_ACC, 160), F32)],
        scratch_types=[pltpu.VMEM((1, CHUNK), jnp.int32),
                       pltpu.VMEM((CHUNK, 160), F32),
                       pltpu.VMEM_SHARED((N_ACC, 160), F32)],
        compiler_params=_SC_PARAMS,
    )
    def k(p0_hbm, p1_hbm, d_hbm, z_hbm, a0_hbm, a1_hbm, ibuf, pbuf, acc):
        cid = lax.axis_index("c")
        sid = lax.axis_index("s")
        r0 = sid * rows_per_tile
        base = sid * per_tile
        pltpu.sync_copy(z_hbm.at[pl.ds(r0, rows_per_tile)],
                        acc.at[pl.ds(r0, rows_per_tile)])
        plsc.subcore_barrier()

        def body(p_hbm):
            @pl.loop(0, n_chunks)
            def _(j):
                pltpu.sync_copy(d_hbm.at[pl.ds(sid * n_chunks + j, 1)],
                                ibuf)
                pltpu.sync_copy(p_hbm.at[pl.ds(base + j * CHUNK, CHUNK)],
                                pbuf)
                pltpu.sync_copy(pbuf, acc.at[ibuf.at[0]], add=True)

        @pl.when(cid == 0)
        def _():
            body(p0_hbm)

        @pl.when(cid == 1)
        def _():
            body(p1_hbm)

        plsc.subcore_barrier()

        @pl.when(cid == 0)
        def _():
            pltpu.sync_copy(acc.at[pl.ds(r0, rows_per_tile)],
                            a0_hbm.at[pl.ds(r0, rows_per_tile)])

        @pl.when(cid == 1)
        def _():
            pltpu.sync_copy(acc.at[pl.ds(r0, rows_per_tile)],
                            a1_hbm.at[pl.ds(r0, rows_per_tile)])

    return k(p0, p1, dst_sc, zeros160)


def _sc_counts(dl_sc, dg_sc, zeros16):
    rows_per_tile = N_ACC // NS
    per_tile = E_PAD // NS
    n_chunks = per_tile // CHUNK

    @functools.partial(
        pl.kernel, mesh=_mesh(),
        out_type=[jax.ShapeDtypeStruct((N_ACC, 16), F32),
                  jax.ShapeDtypeStruct((N_ACC, 16), F32)],
        scratch_types=[pltpu.VMEM((CHUNK,), jnp.int32),
                       pltpu.VMEM((CHUNK, 16), F32),
                       pltpu.VMEM_SHARED((N_ACC, 16), F32),
                       pltpu.SemaphoreType.DMA],
        compiler_params=_SC_PARAMS,
    )
    def k(dl_hbm, dg_hbm, z_hbm, cl_hbm, cg_hbm, ibuf, ones, acc, sem):
        cid = lax.axis_index("c")
        sid = lax.axis_index("s")

        @pl.loop(0, CHUNK)
        def _(r):
            ones[r] = jnp.ones((16,), F32)

        r0 = sid * rows_per_tile
        pltpu.sync_copy(z_hbm.at[pl.ds(r0, rows_per_tile)],
                        acc.at[pl.ds(r0, rows_per_tile)])
        plsc.subcore_barrier()

        def body(d_hbm):
            @pl.loop(0, n_chunks)
            def _(j):
                pltpu.sync_copy(d_hbm.at[sid * n_chunks + j], ibuf)
                pltpu.sync_copy(ones, acc.at[ibuf], add=True)

        @pl.when(cid == 0)
        def _():
            body(dl_hbm)

        @pl.when(cid == 1)
        def _():
            body(dg_hbm)

        plsc.subcore_barrier()

        @pl.when(cid == 0)
        def _():
            pltpu.sync_copy(acc.at[pl.ds(r0, rows_per_tile)],
                            cl_hbm.at[pl.ds(r0, rows_per_tile)])

        @pl.when(cid == 1)
        def _():
            pltpu.sync_copy(acc.at[pl.ds(r0, rows_per_tile)],
                            cg_hbm.at[pl.ds(r0, rows_per_tile)])

    return k(dl_sc, dg_sc, zeros16)


def _tc_seg_sums(pos8, batch2d):
    BLK = 2000
    grid = NND // BLK

    def body(pos_ref, b_ref, sum_ref, cnt_ref):
        i = pl.program_id(0)
        iota = lax.broadcasted_iota(jnp.int32, (NGR, BLK), 0)
        mask = (b_ref[0] == iota).astype(F32)
        psum = jnp.dot(mask, pos_ref[...], preferred_element_type=F32,
                       precision=PREC)
        csum = jnp.broadcast_to(jnp.sum(mask, axis=1, keepdims=True),
                                (NGR, 8))

        @pl.when(i == 0)
        def _():
            sum_ref[...] = jnp.zeros((NGR, 8), F32)
            cnt_ref[...] = jnp.zeros((NGR, 8), F32)

        sum_ref[...] += psum
        cnt_ref[...] += csum

    return pl.pallas_call(
        body,
        grid=(grid,),
        in_specs=[pl.BlockSpec((BLK, 8), lambda i: (i, 0)),
                  pl.BlockSpec((1, 1, BLK), lambda i: (i, 0, 0))],
        out_specs=[pl.BlockSpec((NGR, 8), lambda i: (0, 0)),
                   pl.BlockSpec((NGR, 8), lambda i: (0, 0))],
        out_shape=[jax.ShapeDtypeStruct((NGR, 8), F32),
                   jax.ShapeDtypeStruct((NGR, 8), F32)],
        compiler_params=pltpu.CompilerParams(
            dimension_semantics=("arbitrary",)),
    )(pos8, batch2d)


def _tc_prelude(x, t8, pos8, batch2d, possum, cntg, wt8, bt, wa, ba, wat,
                bat):
    BLK = 2000
    grid = NND // BLK

    def body(x_ref, t_ref, pos_ref, b_ref, ps_ref, cg_ref, wt_ref, bt_ref,
             wa_ref, ba_ref, wat_ref, bat_ref, posc8_ref, posc16_ref,
             s0_ref):
        iota = lax.broadcasted_iota(jnp.int32, (NGR, BLK), 0)
        mask = (b_ref[0] == iota).astype(F32)
        inv = 1.0 / jnp.maximum(cg_ref[...][:, 0:1], 1.0)
        mean = ps_ref[...] * inv
        posc = pos_ref[...] - lax.dot_general(
            mask, mean, (((0,), (0,)), ((), ())),
            preferred_element_type=F32, precision=PREC)
        posc8_ref[...] = posc
        posc16_ref[...] = jnp.concatenate(
            [posc, jnp.zeros((BLK, 8), F32)], axis=1)
        tn8 = lax.dot_general(mask, t_ref[...], (((0,), (0,)), ((), ())),
                              preferred_element_type=F32, precision=PREC)
        ta = jnp.dot(tn8, wt_ref[...], preferred_element_type=F32,
                     precision=PREC) + bt_ref[...]
        sa = jnp.dot(x_ref[...], wa_ref[...], preferred_element_type=F32,
                     precision=PREC) + ba_ref[...]
        s0_ref[...] = jnp.dot(sa + ta, wat_ref[...],
                              preferred_element_type=F32,
                              precision=PREC) + bat_ref[...]

    return pl.pallas_call(
        body,
        grid=(grid,),
        in_specs=[pl.BlockSpec((BLK, ATY), lambda i: (i, 0)),
                  pl.BlockSpec((NGR, 8), lambda i: (0, 0)),
                  pl.BlockSpec((BLK, 8), lambda i: (i, 0)),
                  pl.BlockSpec((1, 1, BLK), lambda i: (i, 0, 0)),
                  pl.BlockSpec((NGR, 8), lambda i: (0, 0)),
                  pl.BlockSpec((NGR, 8), lambda i: (0, 0)),
                  pl.BlockSpec((8, SDIM), lambda i: (0, 0)),
                  pl.BlockSpec((1, SDIM), lambda i: (0, 0)),
                  pl.BlockSpec((ATY, SDIM), lambda i: (0, 0)),
                  pl.BlockSpec((1, SDIM), lambda i: (0, 0)),
                  pl.BlockSpec((SDIM, SDIM), lambda i: (0, 0)),
                  pl.BlockSpec((1, SDIM), lambda i: (0, 0))],
        out_specs=[pl.BlockSpec((BLK, 8), lambda i: (i, 0)),
                   pl.BlockSpec((BLK, 16), lambda i: (i, 0)),
                   pl.BlockSpec((BLK, SDIM), lambda i: (i, 0))],
        out_shape=[jax.ShapeDtypeStruct((NND, 8), F32),
                   jax.ShapeDtypeStruct((NND, 16), F32),
                   jax.ShapeDtypeStruct((NND, SDIM), F32)],
        compiler_params=pltpu.CompilerParams(
            dimension_semantics=("parallel",)),
    )(x, t8, pos8, batch2d, possum, cntg, wt8, bt, wa, ba, wat, bat)


def _tc_attrs(ps16, pd16):
    BLK = 2048
    grid = E_PAD // BLK

    def body(ps_ref, pd_ref, rbfa_ref, rn_ref):
        ps = ps_ref[...][:, :8]
        pd = pd_ref[...][:, :8]
        r = pd - ps
        d2 = jnp.sum(r * r, axis=1, keepdims=True)
        d = jnp.sqrt(jnp.clip(d2, 1e-6, None))
        rn_ref[...] = r / d
        a = jnp.sum(pd * ps, axis=1, keepdims=True)
        mus = lax.broadcasted_iota(jnp.int32, (1, RBFD), 1).astype(F32) * (
            CUT / (RBFD - 1))
        gamma = (CUT / RBFD) ** 2
        rbf = jnp.exp(-((d - mus) ** 2) / gamma)
        rbfa_ref[...] = jnp.concatenate(
            [rbf, a, jnp.zeros((BLK, 31), F32)], axis=1)

    return pl.pallas_call(
        body,
        grid=(grid,),
        in_specs=[pl.BlockSpec((BLK, 16), lambda i: (i, 0)),
                  pl.BlockSpec((BLK, 16), lambda i: (i, 0))],
        out_specs=[pl.BlockSpec((BLK, 64), lambda i: (i, 0)),
                   pl.BlockSpec((BLK, 8), lambda i: (i, 0))],
        out_shape=[jax.ShapeDtypeStruct((E_PAD, 64), F32),
                   jax.ShapeDtypeStruct((E_PAD, 8), F32)],
        compiler_params=pltpu.CompilerParams(
            dimension_semantics=("parallel",)),
    )(ps16, pd16)


def _tc_edge_mm(sd, ss, rbfa, rn, w1, w2, w3, b2):
    BLK = 2048
    grid = E_PAD // BLK

    def body(sd_ref, ss_ref, rb_ref, rn_ref, w1_ref, w2_ref, w3_ref, b_ref,
             p0_ref, p1_ref):
        f = (jnp.dot(sd_ref[...], w1_ref[...], preferred_element_type=F32,
                     precision=PREC)
             + jnp.dot(ss_ref[...], w2_ref[...], preferred_element_type=F32,
                       precision=PREC)
             + jnp.dot(rb_ref[...], w3_ref[...], preferred_element_type=F32,
                       precision=PREC)
             + b_ref[...])
        m = f * jax.nn.sigmoid(f)
        ms = m[:, :SDIM]
        mv = m[:, SDIM:]
        rn = rn_ref[...]
        mv0 = mv * rn[:, 0:1]
        mv1 = mv * rn[:, 1:2]
        mv2 = mv * rn[:, 2:3]
        p0_ref[...] = jnp.concatenate([ms, mv0[:, :32]], axis=1)
        p1_ref[...] = jnp.concatenate([mv0[:, 32:], mv1, mv2], axis=1)

    return pl.pallas_call(
        body,
        grid=(grid,),
        in_specs=[pl.BlockSpec((BLK, SDIM), lambda i: (i, 0)),
                  pl.BlockSpec((BLK, SDIM), lambda i: (i, 0)),
                  pl.BlockSpec((BLK, 64), lambda i: (i, 0)),
                  pl.BlockSpec((BLK, 8), lambda i: (i, 0)),
                  pl.BlockSpec((SDIM, 192), lambda i: (0, 0)),
                  pl.BlockSpec((SDIM, 192), lambda i: (0, 0)),
                  pl.BlockSpec((64, 192), lambda i: (0, 0)),
                  pl.BlockSpec((1, 192), lambda i: (0, 0))],
        out_specs=[pl.BlockSpec((BLK, 160), lambda i: (i, 0)),
                   pl.BlockSpec((BLK, 160), lambda i: (i, 0))],
        out_shape=[jax.ShapeDtypeStruct((E_PAD, 160), F32),
                   jax.ShapeDtypeStruct((E_PAD, 160), F32)],
        compiler_params=pltpu.CompilerParams(
            dimension_semantics=("parallel",)),
    )(sd, ss, rbfa, rn, w1, w2, w3, b2)


def _tc_update(s, v, a0, a1, cnt):
    BLK = 1000
    grid = NND // BLK

    def body(s_ref, v_ref, a0_ref, a1_ref, c_ref, so_ref, vo_ref):
        inv = 1.0 / jnp.maximum(c_ref[...][:, 0:1], 1.0)
        a0v = a0_ref[...]
        so_ref[...] = s_ref[...] + a0v[:, :SDIM] * inv
        vo_ref[...] = v_ref[...] + jnp.concatenate(
            [a0v[:, SDIM:], a1_ref[...]], axis=1) * inv

    return pl.pallas_call(
        body,
        grid=(grid,),
        in_specs=[pl.BlockSpec((BLK, SDIM), lambda i: (i, 0)),
                  pl.BlockSpec((BLK, 192), lambda i: (i, 0)),
                  pl.BlockSpec((BLK, 160), lambda i: (i, 0)),
                  pl.BlockSpec((BLK, 160), lambda i: (i, 0)),
                  pl.BlockSpec((BLK, 16), lambda i: (i, 0))],
        out_specs=[pl.BlockSpec((BLK, SDIM), lambda i: (i, 0)),
                   pl.BlockSpec((BLK, 192), lambda i: (i, 0))],
        out_shape=[jax.ShapeDtypeStruct((NND, SDIM), F32),
                   jax.ShapeDtypeStruct((NND, 192), F32)],
        compiler_params=pltpu.CompilerParams(
            dimension_semantics=("parallel",)),
    )(s, v, a0, a1, cnt)


def _tc_head_a(s, wsh, bsh2, wa, ba2, wf):
    BLK = 2000
    grid = NND // BLK

    def body(s_ref, wsh_ref, bsh_ref, wa_ref, ba_ref, wf_ref,
             atoms_ref, g_ref):
        h = (jnp.dot(s_ref[...], wsh_ref[...], preferred_element_type=F32,
                     precision=PREC) + bsh_ref[...])
        s2 = h * jax.nn.sigmoid(h)
        atoms_ref[...] = jnp.dot(s2, wa_ref[...], preferred_element_type=F32,
                                 precision=PREC) + ba_ref[...]
        g_ref[...] = jnp.dot(s2, wf_ref[...], preferred_element_type=F32,
                             precision=PREC)

    return pl.pallas_call(
        body,
        grid=(grid,),
        in_specs=[pl.BlockSpec((BLK, SDIM), lambda i: (i, 0)),
                  pl.BlockSpec((SDIM, SDIM), lambda i: (0, 0)),
                  pl.BlockSpec((1, SDIM), lambda i: (0, 0)),
                  pl.BlockSpec((SDIM, ATY), lambda i: (0, 0)),
                  pl.BlockSpec((1, ATY), lambda i: (0, 0)),
                  pl.BlockSpec((SDIM, SDIM), lambda i: (0, 0))],
        out_specs=[pl.BlockSpec((BLK, ATY), lambda i: (i, 0)),
                   pl.BlockSpec((BLK, SDIM), lambda i: (i, 0))],
        out_shape=[jax.ShapeDtypeStruct((NND, ATY), F32),
                   jax.ShapeDtypeStruct((NND, SDIM), F32)],
        compiler_params=pltpu.CompilerParams(
            dimension_semantics=("parallel",)),
    )(s, wsh, bsh2, wa, ba2, wf)


def _tc_vhead_sums(v, batch2d, wc_big):
    BLK = 2000
    grid = NND // BLK

    def body(v_ref, b_ref, wc_ref, cp0_ref, sum_ref):
        i = pl.program_id(0)
        cp0 = jnp.dot(v_ref[...], wc_ref[...], preferred_element_type=F32,
                      precision=PREC)
        cp0_ref[...] = cp0
        iota = lax.broadcasted_iota(jnp.int32, (NGR, BLK), 0)
        mask = (b_ref[0] == iota).astype(F32)
        psum = jnp.dot(mask, cp0, preferred_element_type=F32,
                       precision=PREC)

        @pl.when(i == 0)
        def _():
            sum_ref[...] = jnp.zeros((NGR, 8), F32)

        sum_ref[...] += psum

    return pl.pallas_call(
        body,
        grid=(grid,),
        in_specs=[pl.BlockSpec((BLK, 192), lambda i: (i, 0)),
                  pl.BlockSpec((1, 1, BLK), lambda i: (i, 0, 0)),
                  pl.BlockSpec((192, 8), lambda i: (0, 0))],
        out_specs=[pl.BlockSpec((BLK, 8), lambda i: (i, 0)),
                   pl.BlockSpec((NGR, 8), lambda i: (0, 0))],
        out_shape=[jax.ShapeDtypeStruct((NND, 8), F32),
                   jax.ShapeDtypeStruct((NGR, 8), F32)],
        compiler_params=pltpu.CompilerParams(
            dimension_semantics=("arbitrary",)),
    )(v, batch2d, wc_big)


def _tc_coords(cp0, posc8, batch2d, cpsum, cntg):
    BLK = 2000
    grid = NND // BLK

    def body(cp_ref, pc_ref, b_ref, ps_ref, cg_ref, c8_ref, c16_ref):
        iota = lax.broadcasted_iota(jnp.int32, (NGR, BLK), 0)
        mask = (b_ref[0] == iota).astype(F32)
        inv = 1.0 / jnp.maximum(cg_ref[...][:, 0:1], 1.0)
        mean = ps_ref[...] * inv
        coords = pc_ref[...] + cp_ref[...] - lax.dot_general(
            mask, mean, (((0,), (0,)), ((), ())),
            preferred_element_type=F32, precision=PREC)
        c8_ref[...] = coords
        c16_ref[...] = jnp.concatenate(
            [coords, jnp.zeros((BLK, 8), F32)], axis=1)

    return pl.pallas_call(
        body,
        grid=(grid,),
        in_specs=[pl.BlockSpec((BLK, 8), lambda i: (i, 0)),
                  pl.BlockSpec((BLK, 8), lambda i: (i, 0)),
                  pl.BlockSpec((1, 1, BLK), lambda i: (i, 0, 0)),
                  pl.BlockSpec((NGR, 8), lambda i: (0, 0)),
                  pl.BlockSpec((NGR, 8), lambda i: (0, 0))],
        out_specs=[pl.BlockSpec((BLK, 8), lambda i: (i, 0)),
                   pl.BlockSpec((BLK, 16), lambda i: (i, 0))],
        out_shape=[jax.ShapeDtypeStruct((NND, 8), F32),
                   jax.ShapeDtypeStruct((NND, 16), F32)],
        compiler_params=pltpu.CompilerParams(
            dimension_semantics=("parallel",)),
    )(cp0, posc8, batch2d, cpsum, cntg)


def _tc_bond(gi, gj, ci, cj, wdd, bb0, wb1p, bb1p):
    BLK = 2048
    grid = E_PAD // BLK

    def body(gi_ref, gj_ref, ci_ref, cj_ref, wdd_ref, bb0_ref, wb1_ref,
             bb1_ref, out_ref):
        diff = ci_ref[...] - cj_ref[...]
        dd2 = jnp.sum(diff * diff, axis=1, keepdims=True)
        dd = jnp.sqrt(jnp.clip(dd2, 1e-12, None))
        h = gi_ref[...] + gj_ref[...] + dd * wdd_ref[...] + bb0_ref[...]
        h = h * jax.nn.sigmoid(h)
        out_ref[...] = jnp.dot(h, wb1_ref[...], preferred_element_type=F32,
                               precision=PREC) + bb1_ref[...]

    return pl.pallas_call(
        body,
        grid=(grid,),
        in_specs=[pl.BlockSpec((BLK, SDIM), lambda i: (i, 0)),
                  pl.BlockSpec((BLK, SDIM), lambda i: (i, 0)),
                  pl.BlockSpec((BLK, 16), lambda i: (i, 0)),
                  pl.BlockSpec((BLK, 16), lambda i: (i, 0)),
                  pl.BlockSpec((1, SDIM), lambda i: (0, 0)),
                  pl.BlockSpec((1, SDIM), lambda i: (0, 0)),
                  pl.BlockSpec((SDIM, 8), lambda i: (0, 0)),
                  pl.BlockSpec((1, 8), lambda i: (0, 0))],
        out_specs=[pl.BlockSpec((BLK, 8), lambda i: (i, 0))],
        out_shape=[jax.ShapeDtypeStruct((E_PAD, 8), F32)],
        compiler_params=pltpu.CompilerParams(
            dimension_semantics=("parallel",)),
    )(gi, gj, ci, cj, wdd, bb0, wb1p, bb1p)[0]


def kernel(x, t, pos, edge_index_local, edge_index_global, batch, params):
    p = params
    src_l = edge_index_local[0]
    dst_l = edge_index_local[1]
    src_g = edge_index_global[0]
    dst_g = edge_index_global[1]
    pad_e = E_PAD - NED

    def pad0(a):
        return jnp.concatenate([a.astype(jnp.int32),
                                jnp.zeros((pad_e,), jnp.int32)])

    def padd(a):
        return jnp.concatenate([a.astype(jnp.int32),
                                jnp.full((pad_e,), DUMMY, jnp.int32)])

    src_l_g = pad0(src_l).reshape(E_PAD // CHUNK, CHUNK)
    dst_l_g = pad0(dst_l).reshape(E_PAD // CHUNK, CHUNK)
    src_g_g = pad0(src_g).reshape(E_PAD // CHUNK, CHUNK)
    dst_g_g = pad0(dst_g).reshape(E_PAD // CHUNK, CHUNK)
    dst_l_s = padd(dst_l).reshape(E_PAD // CHUNK, CHUNK)
    dst_g_s = padd(dst_g).reshape(E_PAD // CHUNK, CHUNK)

    pos8 = jnp.pad(pos, ((0, 0), (0, 5)))
    t8 = jnp.pad(t, ((0, 0), (0, 7)))
    batch2d = batch.astype(jnp.int32).reshape(NND // 2000, 1, 2000)
    wt8 = jnp.pad(p['W_time'], ((0, 7), (0, 0)))
    bt2 = p['b_time'].reshape(1, SDIM)
    ba2 = p['b_atom'].reshape(1, SDIM)
    bat2 = p['b_at'].reshape(1, SDIM)
    zeros160 = jnp.zeros((N_ACC, 160), F32)
    zeros16 = jnp.zeros((N_ACC, 16), F32)

    possum, cntg = _tc_seg_sums(pos8, batch2d)
    posc8, posc16, s = _tc_prelude(
        x, t8, pos8, batch2d, possum, cntg, wt8, bt2, p['W_atom'], ba2,
        p['W_at'], bat2)

    cnt_l, cnt_g = _sc_counts(dst_l_s, dst_g_s, zeros16)

    ps_l, pd_l = _sc_gather(posc16, src_l_g, dst_l_g, 16)
    ps_g, pd_g = _sc_gather(posc16, src_g_g, dst_g_g, 16)
    rbfa_l, rn_l = _tc_attrs(ps_l, pd_l)
    rbfa_g, rn_g = _tc_attrs(ps_g, pd_g)

    v = jnp.zeros((NND, 192), F32)
    for l in range(3):
        lp = p['layers'][l]
        for (dg, sg, ds, rbfa, rn, cnt, w, b) in (
                (dst_l_g, src_l_g, dst_l_s, rbfa_l, rn_l, cnt_l,
                 lp['Wl'], lp['bl']),
                (dst_g_g, src_g_g, dst_g_s, rbfa_g, rn_g, cnt_g,
                 lp['Wg'], lp['bg'])):
            sdst, ssrc = _sc_gather(s, dg, sg, SDIM)
            w1 = w[:SDIM]
            w2 = w[SDIM:2 * SDIM]
            w3 = jnp.pad(w[2 * SDIM:], ((0, 31), (0, 0)))
            b2 = b.reshape(1, -1)
            p0, p1 = _tc_edge_mm(sdst, ssrc, rbfa, rn, w1, w2, w3, b2)
            a0, a1 = _sc_scatter_pair(p0, p1, ds, zeros160)
            s, v = _tc_update(s, v, a0, a1, cnt)

    wf = p['W_b0'][:SDIM]
    wdd = p['W_b0'][SDIM:SDIM + 1]
    bsh2 = p['b_sh'].reshape(1, SDIM)
    bab2 = p['b_a'].reshape(1, ATY)
    atoms16, gtab = _tc_head_a(s, p['W_sh'], bsh2, p['W_a'], bab2, wf)

    wc = p['W_c'][:, 0]
    wc_big = jnp.zeros((192, 8), F32)
    wc_big = wc_big.at[0:64, 0].set(wc)
    wc_big = wc_big.at[64:128, 1].set(wc)
    wc_big = wc_big.at[128:192, 2].set(wc)
    cp0, cpsum = _tc_vhead_sums(v, batch2d, wc_big)
    coords8, coords16 = _tc_coords(cp0, posc8, batch2d, cpsum, cntg)

    g_j, g_i = _sc_gather(gtab, src_g_g, dst_g_g, SDIM)
    c_j, c_i = _sc_gather(coords16, src_g_g, dst_g_g, 16)

    bb0 = p['b_b0'].reshape(1, SDIM)
    wb1p = jnp.pad(p['W_b1'], ((0, 0), (0, 8 - BTY)))
    bb1p = jnp.pad(p['b_b1'], ((0, 8 - BTY))).reshape(1, 8)
    bonds8 = _tc_bond(g_i, g_j, c_i, c_j, wdd, bb0, wb1p, bb1p)

    return coords8[:, :3], atoms16, bonds8[:NED, :BTY]


# speedup vs baseline: 13.9462x; 1.0315x over previous
import functools

import jax
import jax.numpy as jnp
from jax import lax
from jax.experimental import pallas as pl
from jax.experimental.pallas import tpu as pltpu
from jax.experimental.pallas import tpu_sc as plsc

NND = 10000
NED = 160000
NGR = 256
SDIM = 128
VDIM = 64
RBFD = 32
ATY = 16
BTY = 5
CUT = 7.5

E_PAD = 163840
N_ACC = 10240
DUMMY = NND
NC, NS = 2, 16
CHUNK = 128
SCH = 64

PREC = lax.Precision.HIGHEST
F32 = jnp.float32


def _mesh():
    return plsc.VectorSubcoreMesh(core_axis_name="c", subcore_axis_name="s")


_SC_PARAMS = pltpu.CompilerParams(use_tc_tiling_on_sc=False)


def _sc_gather(table, idx0, idx1, d):
    per_tile = E_PAD // (NC * NS)
    n_chunks = per_tile // CHUNK
    rows = table.shape[0]
    rows_pc = rows // NS

    @functools.partial(
        pl.kernel, mesh=_mesh(),
        out_type=[jax.ShapeDtypeStruct((E_PAD, d), F32),
                  jax.ShapeDtypeStruct((E_PAD, d), F32)],
        scratch_types=[pltpu.VMEM((2, n_chunks, CHUNK), jnp.int32),
                       pltpu.VMEM((CHUNK, d), F32),
                       pltpu.VMEM((CHUNK, d), F32),
                       pltpu.VMEM_SHARED((rows, d), F32),
                       pltpu.SemaphoreType.DMA,
                       pltpu.SemaphoreType.DMA,
                       pltpu.SemaphoreType.DMA,
                       pltpu.SemaphoreType.DMA],
        compiler_params=_SC_PARAMS,
    )
    def k(table_hbm, i0_hbm, i1_hbm, o0_hbm, o1_hbm, idxv, r0, r1, tab,
          g0, g1, w0, w1):
        wid = lax.axis_index("s") * NC + lax.axis_index("c")
        sid = lax.axis_index("s")
        base = wid * per_tile
        pltpu.sync_copy(i0_hbm.at[pl.ds(wid * n_chunks, n_chunks)],
                        idxv.at[0])
        pltpu.sync_copy(i1_hbm.at[pl.ds(wid * n_chunks, n_chunks)],
                        idxv.at[1])
        pltpu.sync_copy(table_hbm.at[pl.ds(sid * rows_pc, rows_pc)],
                        tab.at[pl.ds(sid * rows_pc, rows_pc)])
        plsc.subcore_barrier()

        bufs = (r0, r1)
        gsems = (g0, g1)
        wsems = (w0, w1)
        outs = (o0_hbm, o1_hbm)
        gh = {}
        wh = {}
        for j in range(n_chunks):
            for s_ in (0, 1):
                if j > 0:
                    wh[s_].wait()
                gh[s_] = pltpu.async_copy(tab.at[idxv.at[s_, j]],
                                          bufs[s_], gsems[s_])
            for s_ in (0, 1):
                gh[s_].wait()
                wh[s_] = pltpu.async_copy(
                    bufs[s_],
                    outs[s_].at[pl.ds(base + j * CHUNK, CHUNK)],
                    wsems[s_])
        wh[0].wait()
        wh[1].wait()

    return k(table, idx0, idx1)


def _sc_scatter_pair(p0, p1, dst_sc, zeros160):
    rows_per_tile = N_ACC // NS
    per_tile = E_PAD // NS
    n_chunks = per_tile // SCH
    half = n_chunks // 2

    @functools.partial(
        pl.kernel, mesh=_mesh(),
        out_type=[jax.ShapeDtypeStruct((N_ACC, 160), F32),
                  jax.ShapeDtypeStruct((N_ACC, 160), F32)],
        scratch_types=[pltpu.VMEM((half, SCH), jnp.int32),
                       pltpu.VMEM((SCH, 160), F32),
                       pltpu.VMEM((SCH, 160), F32),
                       pltpu.VMEM_SHARED((N_ACC, 160), F32),
                       pltpu.SemaphoreType.DMA,
                       pltpu.SemaphoreType.DMA,
                       pltpu.SemaphoreType.DMA,
                       pltpu.SemaphoreType.DMA],
        compiler_params=_SC_PARAMS,
    )
    def k(p0_hbm, p1_hbm, d_hbm, z_hbm, a0_hbm, a1_hbm, idxv, pba, pbb,
          acc, la, lb, sa, sb):
        cid = lax.axis_index("c")
        sid = lax.axis_index("s")
        r0 = sid * rows_per_tile
        base = sid * per_tile
        pltpu.sync_copy(z_hbm.at[pl.ds(r0, rows_per_tile)],
                        acc.at[pl.ds(r0, rows_per_tile)])
        plsc.subcore_barrier()

        bufs = (pba, pbb)
        lsems = (la, lb)
        ssems = (sa, sb)

        def body(p_hbm):
            def start_load(j, b):
                return pltpu.async_copy(
                    p_hbm.at[pl.ds(base + j * SCH, SCH)], bufs[b],
                    lsems[b])

            def wait_load(b):
                pltpu.make_async_copy(
                    p_hbm.at[pl.ds(base, SCH)], bufs[b], lsems[b]).wait()

            for ph in (0, 1):
                pltpu.sync_copy(
                    d_hbm.at[pl.ds(sid * n_chunks + ph * half, half)],
                    idxv)
                j0 = ph * half
                start_load(j0, 0)
                start_load(j0 + 1, 1)

                @pl.loop(0, half // 2)
                def _(g):
                    sh = {}
                    for b in (0, 1):
                        wait_load(b)
                        sh[b] = pltpu.async_copy(
                            bufs[b], acc.at[idxv.at[2 * g + b]],
                            ssems[b], add=True)
                    for b in (0, 1):
                        sh[b].wait()

                        @pl.when(2 * g + b + 2 < half)
                        def _():
                            pltpu.async_copy(
                                p_hbm.at[pl.ds(
                                    base + (j0 + 2 * g + b + 2) * SCH,
                                    SCH)],
                                bufs[b], lsems[b])

        @pl.when(cid == 0)
        def _():
            body(p0_hbm)

        @pl.when(cid == 1)
        def _():
            body(p1_hbm)

        plsc.subcore_barrier()

        @pl.when(cid == 0)
        def _():
            pltpu.sync_copy(acc.at[pl.ds(r0, rows_per_tile)],
                            a0_hbm.at[pl.ds(r0, rows_per_tile)])

        @pl.when(cid == 1)
        def _():
            pltpu.sync_copy(acc.at[pl.ds(r0, rows_per_tile)],
                            a1_hbm.at[pl.ds(r0, rows_per_tile)])

    return k(p0, p1, dst_sc, zeros160)


def _sc_counts(dl_sc, dg_sc, zeros16):
    rows_per_tile = N_ACC // NS
    per_tile = E_PAD // NS
    n_chunks = per_tile // CHUNK

    @functools.partial(
        pl.kernel, mesh=_mesh(),
        out_type=[jax.ShapeDtypeStruct((N_ACC, 16), F32),
                  jax.ShapeDtypeStruct((N_ACC, 16), F32)],
        scratch_types=[pltpu.VMEM((CHUNK,), jnp.int32),
                       pltpu.VMEM((CHUNK, 16), F32),
                       pltpu.VMEM_SHARED((N_ACC, 16), F32),
                       pltpu.SemaphoreType.DMA],
        compiler_params=_SC_PARAMS,
    )
    def k(dl_hbm, dg_hbm, z_hbm, cl_hbm, cg_hbm, ibuf, ones, acc, sem):
        cid = lax.axis_index("c")
        sid = lax.axis_index("s")

        @pl.loop(0, CHUNK)
        def _(r):
            ones[r] = jnp.ones((16,), F32)

        r0 = sid * rows_per_tile
        pltpu.sync_copy(z_hbm.at[pl.ds(r0, rows_per_tile)],
                        acc.at[pl.ds(r0, rows_per_tile)])
        plsc.subcore_barrier()

        def body(d_hbm):
            @pl.loop(0, n_chunks)
            def _(j):
                pltpu.sync_copy(d_hbm.at[sid * n_chunks + j], ibuf)
                pltpu.sync_copy(ones, acc.at[ibuf], add=True)

        @pl.when(cid == 0)
        def _():
            body(dl_hbm)

        @pl.when(cid == 1)
        def _():
            body(dg_hbm)

        plsc.subcore_barrier()

        @pl.when(cid == 0)
        def _():
            pltpu.sync_copy(acc.at[pl.ds(r0, rows_per_tile)],
                            cl_hbm.at[pl.ds(r0, rows_per_tile)])

        @pl.when(cid == 1)
        def _():
            pltpu.sync_copy(acc.at[pl.ds(r0, rows_per_tile)],
                            cg_hbm.at[pl.ds(r0, rows_per_tile)])

    return k(dl_sc, dg_sc, zeros16)


def _tc_seg_sums(pos8, batch2d):
    BLK = 2000
    grid = NND // BLK

    def body(pos_ref, b_ref, sum_ref, cnt_ref):
        i = pl.program_id(0)
        iota = lax.broadcasted_iota(jnp.int32, (NGR, BLK), 0)
        mask = (b_ref[0] == iota).astype(F32)
        psum = jnp.dot(mask, pos_ref[...], preferred_element_type=F32,
                       precision=PREC)
        csum = jnp.broadcast_to(jnp.sum(mask, axis=1, keepdims=True),
                                (NGR, 8))

        @pl.when(i == 0)
        def _():
            sum_ref[...] = jnp.zeros((NGR, 8), F32)
            cnt_ref[...] = jnp.zeros((NGR, 8), F32)

        sum_ref[...] += psum
        cnt_ref[...] += csum

    return pl.pallas_call(
        body,
        grid=(grid,),
        in_specs=[pl.BlockSpec((BLK, 8), lambda i: (i, 0)),
                  pl.BlockSpec((1, 1, BLK), lambda i: (i, 0, 0))],
        out_specs=[pl.BlockSpec((NGR, 8), lambda i: (0, 0)),
                   pl.BlockSpec((NGR, 8), lambda i: (0, 0))],
        out_shape=[jax.ShapeDtypeStruct((NGR, 8), F32),
                   jax.ShapeDtypeStruct((NGR, 8), F32)],
        compiler_params=pltpu.CompilerParams(
            dimension_semantics=("arbitrary",)),
    )(pos8, batch2d)


def _tc_prelude(x, t8, pos8, batch2d, possum, cntg, wt8, bt, wa, ba, wat,
                bat):
    BLK = 2000
    grid = NND // BLK

    def body(x_ref, t_ref, pos_ref, b_ref, ps_ref, cg_ref, wt_ref, bt_ref,
             wa_ref, ba_ref, wat_ref, bat_ref, posc8_ref, posc16_ref,
             s0_ref):
        iota = lax.broadcasted_iota(jnp.int32, (NGR, BLK), 0)
        mask = (b_ref[0] == iota).astype(F32)
        inv = 1.0 / jnp.maximum(cg_ref[...][:, 0:1], 1.0)
        mean = ps_ref[...] * inv
        posc = pos_ref[...] - lax.dot_general(
            mask, mean, (((0,), (0,)), ((), ())),
            preferred_element_type=F32, precision=PREC)
        posc8_ref[...] = posc
        posc16_ref[...] = jnp.concatenate(
            [posc, jnp.zeros((BLK, 8), F32)], axis=1)
        tn8 = lax.dot_general(mask, t_ref[...], (((0,), (0,)), ((), ())),
                              preferred_element_type=F32, precision=PREC)
        ta = jnp.dot(tn8, wt_ref[...], preferred_element_type=F32,
                     precision=PREC) + bt_ref[...]
        sa = jnp.dot(x_ref[...], wa_ref[...], preferred_element_type=F32,
                     precision=PREC) + ba_ref[...]
        s0_ref[...] = jnp.dot(sa + ta, wat_ref[...],
                              preferred_element_type=F32,
                              precision=PREC) + bat_ref[...]

    return pl.pallas_call(
        body,
        grid=(grid,),
        in_specs=[pl.BlockSpec((BLK, ATY), lambda i: (i, 0)),
                  pl.BlockSpec((NGR, 8), lambda i: (0, 0)),
                  pl.BlockSpec((BLK, 8), lambda i: (i, 0)),
                  pl.BlockSpec((1, 1, BLK), lambda i: (i, 0, 0)),
                  pl.BlockSpec((NGR, 8), lambda i: (0, 0)),
                  pl.BlockSpec((NGR, 8), lambda i: (0, 0)),
                  pl.BlockSpec((8, SDIM), lambda i: (0, 0)),
                  pl.BlockSpec((1, SDIM), lambda i: (0, 0)),
                  pl.BlockSpec((ATY, SDIM), lambda i: (0, 0)),
                  pl.BlockSpec((1, SDIM), lambda i: (0, 0)),
                  pl.BlockSpec((SDIM, SDIM), lambda i: (0, 0)),
                  pl.BlockSpec((1, SDIM), lambda i: (0, 0))],
        out_specs=[pl.BlockSpec((BLK, 8), lambda i: (i, 0)),
                   pl.BlockSpec((BLK, 16), lambda i: (i, 0)),
                   pl.BlockSpec((BLK, SDIM), lambda i: (i, 0))],
        out_shape=[jax.ShapeDtypeStruct((NND, 8), F32),
                   jax.ShapeDtypeStruct((NND, 16), F32),
                   jax.ShapeDtypeStruct((NND, SDIM), F32)],
        compiler_params=pltpu.CompilerParams(
            dimension_semantics=("parallel",)),
    )(x, t8, pos8, batch2d, possum, cntg, wt8, bt, wa, ba, wat, bat)


def _tc_attrs(ps16, pd16):
    BLK = 2048
    grid = E_PAD // BLK

    def body(ps_ref, pd_ref, rbfa_ref, rn_ref):
        ps = ps_ref[...][:, :8]
        pd = pd_ref[...][:, :8]
        r = pd - ps
        d2 = jnp.sum(r * r, axis=1, keepdims=True)
        d = jnp.sqrt(jnp.clip(d2, 1e-6, None))
        rn_ref[...] = r / d
        a = jnp.sum(pd * ps, axis=1, keepdims=True)
        mus = lax.broadcasted_iota(jnp.int32, (1, RBFD), 1).astype(F32) * (
            CUT / (RBFD - 1))
        gamma = (CUT / RBFD) ** 2
        rbf = jnp.exp(-((d - mus) ** 2) / gamma)
        rbfa_ref[...] = jnp.concatenate(
            [rbf, a, jnp.zeros((BLK, 31), F32)], axis=1)

    return pl.pallas_call(
        body,
        grid=(grid,),
        in_specs=[pl.BlockSpec((BLK, 16), lambda i: (i, 0)),
                  pl.BlockSpec((BLK, 16), lambda i: (i, 0))],
        out_specs=[pl.BlockSpec((BLK, 64), lambda i: (i, 0)),
                   pl.BlockSpec((BLK, 8), lambda i: (i, 0))],
        out_shape=[jax.ShapeDtypeStruct((E_PAD, 64), F32),
                   jax.ShapeDtypeStruct((E_PAD, 8), F32)],
        compiler_params=pltpu.CompilerParams(
            dimension_semantics=("parallel",)),
    )(ps16, pd16)


def _tc_edge_mm(sd, ss, rbfa, rn, w1, w2, w3, b2):
    BLK = 2048
    grid = E_PAD // BLK

    def body(sd_ref, ss_ref, rb_ref, rn_ref, w1_ref, w2_ref, w3_ref, b_ref,
             p0_ref, p1_ref):
        f = (jnp.dot(sd_ref[...], w1_ref[...], preferred_element_type=F32,
                     precision=PREC)
             + jnp.dot(ss_ref[...], w2_ref[...], preferred_element_type=F32,
                       precision=PREC)
             + jnp.dot(rb_ref[...], w3_ref[...], preferred_element_type=F32,
                       precision=PREC)
             + b_ref[...])
        m = f * jax.nn.sigmoid(f)
        ms = m[:, :SDIM]
        mv = m[:, SDIM:]
        rn = rn_ref[...]
        mv0 = mv * rn[:, 0:1]
        mv1 = mv * rn[:, 1:2]
        mv2 = mv * rn[:, 2:3]
        p0_ref[...] = jnp.concatenate([ms, mv0[:, :32]], axis=1)
        p1_ref[...] = jnp.concatenate([mv0[:, 32:], mv1, mv2], axis=1)

    return pl.pallas_call(
        body,
        grid=(grid,),
        in_specs=[pl.BlockSpec((BLK, SDIM), lambda i: (i, 0)),
                  pl.BlockSpec((BLK, SDIM), lambda i: (i, 0)),
                  pl.BlockSpec((BLK, 64), lambda i: (i, 0)),
                  pl.BlockSpec((BLK, 8), lambda i: (i, 0)),
                  pl.BlockSpec((SDIM, 192), lambda i: (0, 0)),
                  pl.BlockSpec((SDIM, 192), lambda i: (0, 0)),
                  pl.BlockSpec((64, 192), lambda i: (0, 0)),
                  pl.BlockSpec((1, 192), lambda i: (0, 0))],
        out_specs=[pl.BlockSpec((BLK, 160), lambda i: (i, 0)),
                   pl.BlockSpec((BLK, 160), lambda i: (i, 0))],
        out_shape=[jax.ShapeDtypeStruct((E_PAD, 160), F32),
                   jax.ShapeDtypeStruct((E_PAD, 160), F32)],
        compiler_params=pltpu.CompilerParams(
            dimension_semantics=("parallel",)),
    )(sd, ss, rbfa, rn, w1, w2, w3, b2)


def _tc_update(s, v, a0, a1, cnt):
    BLK = 1000
    grid = NND // BLK

    def body(s_ref, v_ref, a0_ref, a1_ref, c_ref, so_ref, vo_ref):
        inv = 1.0 / jnp.maximum(c_ref[...][:, 0:1], 1.0)
        a0v = a0_ref[...]
        so_ref[...] = s_ref[...] + a0v[:, :SDIM] * inv
        vo_ref[...] = v_ref[...] + jnp.concatenate(
            [a0v[:, SDIM:], a1_ref[...]], axis=1) * inv

    return pl.pallas_call(
        body,
        grid=(grid,),
        in_specs=[pl.BlockSpec((BLK, SDIM), lambda i: (i, 0)),
                  pl.BlockSpec((BLK, 192), lambda i: (i, 0)),
                  pl.BlockSpec((BLK, 160), lambda i: (i, 0)),
                  pl.BlockSpec((BLK, 160), lambda i: (i, 0)),
                  pl.BlockSpec((BLK, 16), lambda i: (i, 0))],
        out_specs=[pl.BlockSpec((BLK, SDIM), lambda i: (i, 0)),
                   pl.BlockSpec((BLK, 192), lambda i: (i, 0))],
        out_shape=[jax.ShapeDtypeStruct((NND, SDIM), F32),
                   jax.ShapeDtypeStruct((NND, 192), F32)],
        compiler_params=pltpu.CompilerParams(
            dimension_semantics=("parallel",)),
    )(s, v, a0, a1, cnt)


def _tc_head_a(s, wsh, bsh2, wa, ba2, wf):
    BLK = 2000
    grid = NND // BLK

    def body(s_ref, wsh_ref, bsh_ref, wa_ref, ba_ref, wf_ref,
             atoms_ref, g_ref):
        h = (jnp.dot(s_ref[...], wsh_ref[...], preferred_element_type=F32,
                     precision=PREC) + bsh_ref[...])
        s2 = h * jax.nn.sigmoid(h)
        atoms_ref[...] = jnp.dot(s2, wa_ref[...], preferred_element_type=F32,
                                 precision=PREC) + ba_ref[...]
        g_ref[...] = jnp.dot(s2, wf_ref[...], preferred_element_type=F32,
                             precision=PREC)

    return pl.pallas_call(
        body,
        grid=(grid,),
        in_specs=[pl.BlockSpec((BLK, SDIM), lambda i: (i, 0)),
                  pl.BlockSpec((SDIM, SDIM), lambda i: (0, 0)),
                  pl.BlockSpec((1, SDIM), lambda i: (0, 0)),
                  pl.BlockSpec((SDIM, ATY), lambda i: (0, 0)),
                  pl.BlockSpec((1, ATY), lambda i: (0, 0)),
                  pl.BlockSpec((SDIM, SDIM), lambda i: (0, 0))],
        out_specs=[pl.BlockSpec((BLK, ATY), lambda i: (i, 0)),
                   pl.BlockSpec((BLK, SDIM), lambda i: (i, 0))],
        out_shape=[jax.ShapeDtypeStruct((NND, ATY), F32),
                   jax.ShapeDtypeStruct((NND, SDIM), F32)],
        compiler_params=pltpu.CompilerParams(
            dimension_semantics=("parallel",)),
    )(s, wsh, bsh2, wa, ba2, wf)


def _tc_vhead_sums(v, batch2d, wc_big):
    BLK = 2000
    grid = NND // BLK

    def body(v_ref, b_ref, wc_ref, cp0_ref, sum_ref):
        i = pl.program_id(0)
        cp0 = jnp.dot(v_ref[...], wc_ref[...], preferred_element_type=F32,
                      precision=PREC)
        cp0_ref[...] = cp0
        iota = lax.broadcasted_iota(jnp.int32, (NGR, BLK), 0)
        mask = (b_ref[0] == iota).astype(F32)
        psum = jnp.dot(mask, cp0, preferred_element_type=F32,
                       precision=PREC)

        @pl.when(i == 0)
        def _():
            sum_ref[...] = jnp.zeros((NGR, 8), F32)

        sum_ref[...] += psum

    return pl.pallas_call(
        body,
        grid=(grid,),
        in_specs=[pl.BlockSpec((BLK, 192), lambda i: (i, 0)),
                  pl.BlockSpec((1, 1, BLK), lambda i: (i, 0, 0)),
                  pl.BlockSpec((192, 8), lambda i: (0, 0))],
        out_specs=[pl.BlockSpec((BLK, 8), lambda i: (i, 0)),
                   pl.BlockSpec((NGR, 8), lambda i: (0, 0))],
        out_shape=[jax.ShapeDtypeStruct((NND, 8), F32),
                   jax.ShapeDtypeStruct((NGR, 8), F32)],
        compiler_params=pltpu.CompilerParams(
            dimension_semantics=("arbitrary",)),
    )(v, batch2d, wc_big)


def _tc_coords(cp0, posc8, batch2d, cpsum, cntg):
    BLK = 2000
    grid = NND // BLK

    def body(cp_ref, pc_ref, b_ref, ps_ref, cg_ref, c8_ref, c16_ref):
        iota = lax.broadcasted_iota(jnp.int32, (NGR, BLK), 0)
        mask = (b_ref[0] == iota).astype(F32)
        inv = 1.0 / jnp.maximum(cg_ref[...][:, 0:1], 1.0)
        mean = ps_ref[...] * inv
        coords = pc_ref[...] + cp_ref[...] - lax.dot_general(
            mask, mean, (((0,), (0,)), ((), ())),
            preferred_element_type=F32, precision=PREC)
        c8_ref[...] = coords
        c16_ref[...] = jnp.concatenate(
            [coords, jnp.zeros((BLK, 8), F32)], axis=1)

    return pl.pallas_call(
        body,
        grid=(grid,),
        in_specs=[pl.BlockSpec((BLK, 8), lambda i: (i, 0)),
                  pl.BlockSpec((BLK, 8), lambda i: (i, 0)),
                  pl.BlockSpec((1, 1, BLK), lambda i: (i, 0, 0)),
                  pl.BlockSpec((NGR, 8), lambda i: (0, 0)),
                  pl.BlockSpec((NGR, 8), lambda i: (0, 0))],
        out_specs=[pl.BlockSpec((BLK, 8), lambda i: (i, 0)),
                   pl.BlockSpec((BLK, 16), lambda i: (i, 0))],
        out_shape=[jax.ShapeDtypeStruct((NND, 8), F32),
                   jax.ShapeDtypeStruct((NND, 16), F32)],
        compiler_params=pltpu.CompilerParams(
            dimension_semantics=("parallel",)),
    )(cp0, posc8, batch2d, cpsum, cntg)


def _tc_bond(gi, gj, ci, cj, wdd, bb0, wb1p, bb1p):
    BLK = 2048
    grid = E_PAD // BLK

    def body(gi_ref, gj_ref, ci_ref, cj_ref, wdd_ref, bb0_ref, wb1_ref,
             bb1_ref, out_ref):
        diff = ci_ref[...] - cj_ref[...]
        dd2 = jnp.sum(diff * diff, axis=1, keepdims=True)
        dd = jnp.sqrt(jnp.clip(dd2, 1e-12, None))
        h = gi_ref[...] + gj_ref[...] + dd * wdd_ref[...] + bb0_ref[...]
        h = h * jax.nn.sigmoid(h)
        out_ref[...] = jnp.dot(h, wb1_ref[...], preferred_element_type=F32,
                               precision=PREC) + bb1_ref[...]

    return pl.pallas_call(
        body,
        grid=(grid,),
        in_specs=[pl.BlockSpec((BLK, SDIM), lambda i: (i, 0)),
                  pl.BlockSpec((BLK, SDIM), lambda i: (i, 0)),
                  pl.BlockSpec((BLK, 16), lambda i: (i, 0)),
                  pl.BlockSpec((BLK, 16), lambda i: (i, 0)),
                  pl.BlockSpec((1, SDIM), lambda i: (0, 0)),
                  pl.BlockSpec((1, SDIM), lambda i: (0, 0)),
                  pl.BlockSpec((SDIM, 8), lambda i: (0, 0)),
                  pl.BlockSpec((1, 8), lambda i: (0, 0))],
        out_specs=[pl.BlockSpec((BLK, 8), lambda i: (i, 0))],
        out_shape=[jax.ShapeDtypeStruct((E_PAD, 8), F32)],
        compiler_params=pltpu.CompilerParams(
            dimension_semantics=("parallel",)),
    )(gi, gj, ci, cj, wdd, bb0, wb1p, bb1p)[0]


def kernel(x, t, pos, edge_index_local, edge_index_global, batch, params):
    p = params
    src_l = edge_index_local[0]
    dst_l = edge_index_local[1]
    src_g = edge_index_global[0]
    dst_g = edge_index_global[1]
    pad_e = E_PAD - NED

    def pad0(a):
        return jnp.concatenate([a.astype(jnp.int32),
                                jnp.zeros((pad_e,), jnp.int32)])

    def padd(a):
        return jnp.concatenate([a.astype(jnp.int32),
                                jnp.full((pad_e,), DUMMY, jnp.int32)])

    src_l_g = pad0(src_l).reshape(E_PAD // CHUNK, CHUNK)
    dst_l_g = pad0(dst_l).reshape(E_PAD // CHUNK, CHUNK)
    src_g_g = pad0(src_g).reshape(E_PAD // CHUNK, CHUNK)
    dst_g_g = pad0(dst_g).reshape(E_PAD // CHUNK, CHUNK)
    dst_l_c = padd(dst_l).reshape(E_PAD // CHUNK, CHUNK)
    dst_g_c = padd(dst_g).reshape(E_PAD // CHUNK, CHUNK)
    dst_l_s = padd(dst_l).reshape(E_PAD // SCH, SCH)
    dst_g_s = padd(dst_g).reshape(E_PAD // SCH, SCH)

    pos8 = jnp.pad(pos, ((0, 0), (0, 5)))
    t8 = jnp.pad(t, ((0, 0), (0, 7)))
    batch2d = batch.astype(jnp.int32).reshape(NND // 2000, 1, 2000)
    wt8 = jnp.pad(p['W_time'], ((0, 7), (0, 0)))
    bt2 = p['b_time'].reshape(1, SDIM)
    ba2 = p['b_atom'].reshape(1, SDIM)
    bat2 = p['b_at'].reshape(1, SDIM)
    zeros160 = jnp.zeros((N_ACC, 160), F32)
    zeros16 = jnp.zeros((N_ACC, 16), F32)

    possum, cntg = _tc_seg_sums(pos8, batch2d)
    posc8, posc16, s = _tc_prelude(
        x, t8, pos8, batch2d, possum, cntg, wt8, bt2, p['W_atom'], ba2,
        p['W_at'], bat2)

    cnt_l, cnt_g = _sc_counts(dst_l_c, dst_g_c, zeros16)

    ps_l, pd_l = _sc_gather(posc16, src_l_g, dst_l_g, 16)
    ps_g, pd_g = _sc_gather(posc16, src_g_g, dst_g_g, 16)
    rbfa_l, rn_l = _tc_attrs(ps_l, pd_l)
    rbfa_g, rn_g = _tc_attrs(ps_g, pd_g)

    v = jnp.zeros((NND, 192), F32)
    for l in range(3):
        lp = p['layers'][l]
        for (dg, sg, ds, rbfa, rn, cnt, w, b) in (
                (dst_l_g, src_l_g, dst_l_s, rbfa_l, rn_l, cnt_l,
                 lp['Wl'], lp['bl']),
                (dst_g_g, src_g_g, dst_g_s, rbfa_g, rn_g, cnt_g,
                 lp['Wg'], lp['bg'])):
            sdst, ssrc = _sc_gather(s, dg, sg, SDIM)
            w1 = w[:SDIM]
            w2 = w[SDIM:2 * SDIM]
            w3 = jnp.pad(w[2 * SDIM:], ((0, 31), (0, 0)))
            b2 = b.reshape(1, -1)
            p0, p1 = _tc_edge_mm(sdst, ssrc, rbfa, rn, w1, w2, w3, b2)
            a0, a1 = _sc_scatter_pair(p0, p1, ds, zeros160)
            s, v = _tc_update(s, v, a0, a1, cnt)

    wf = p['W_b0'][:SDIM]
    wdd = p['W_b0'][SDIM:SDIM + 1]
    bsh2 = p['b_sh'].reshape(1, SDIM)
    bab2 = p['b_a'].reshape(1, ATY)
    atoms16, gtab = _tc_head_a(s, p['W_sh'], bsh2, p['W_a'], bab2, wf)

    wc = p['W_c'][:, 0]
    wc_big = jnp.zeros((192, 8), F32)
    wc_big = wc_big.at[0:64, 0].set(wc)
    wc_big = wc_big.at[64:128, 1].set(wc)
    wc_big = wc_big.at[128:192, 2].set(wc)
    cp0, cpsum = _tc_vhead_sums(v, batch2d, wc_big)
    coords8, coords16 = _tc_coords(cp0, posc8, batch2d, cpsum, cntg)

    g_j, g_i = _sc_gather(gtab, src_g_g, dst_g_g, SDIM)
    c_j, c_i = _sc_gather(coords16, src_g_g, dst_g_g, 16)

    bb0 = p['b_b0'].reshape(1, SDIM)
    wb1p = jnp.pad(p['W_b1'], ((0, 0), (0, 8 - BTY)))
    bb1p = jnp.pad(p['b_b1'], ((0, 8 - BTY))).reshape(1, 8)
    bonds8 = _tc_bond(g_i, g_j, c_i, c_j, wdd, bb0, wb1p, bb1p)

    return coords8[:, :3], atoms16, bonds8[:NED, :BTY]
